# baseline (speedup 1.0000x reference)
import jax
import jax.numpy as jnp
from jax.experimental import pallas as pl
from jax.experimental.pallas import tpu as pltpu

DIM = 768
HEADS = 12
DIM_HEAD = 64
TOPK = 16
INNER = HEADS * DIM_HEAD
SEQ = 2048
BQ = 1024
_LANE = 128
_NCOL = SEQ // _LANE


def _ln_qkv_kernel(x_ref, g_ref, b_ref, w_ref, q_ref, k_ref, v_ref):
    xb = x_ref[:]
    mu = jnp.mean(xb, axis=-1, keepdims=True)
    var = jnp.mean((xb - mu) ** 2, axis=-1, keepdims=True)
    xn = (xb - mu) * jax.lax.rsqrt(var + 1e-5)
    xn = xn * g_ref[:] + b_ref[:]
    qkv = jax.lax.dot_general(
        xn, w_ref[:], (((1,), (0,)), ((), ())),
        preferred_element_type=jnp.float32)
    ones = jnp.ones((BQ, DIM_HEAD), jnp.float32)
    for h in range(HEADS):
        q_ref[h] = qkv[:, h * DIM_HEAD:(h + 1) * DIM_HEAD]
        k_ref[h] = qkv[:, INNER + h * DIM_HEAD:INNER + (h + 1) * DIM_HEAD]
        vh = qkv[:, 2 * INNER + h * DIM_HEAD:2 * INNER + (h + 1) * DIM_HEAD]
        v_ref[h] = jnp.concatenate([vh, ones], axis=-1)


def _attn_kernel(alpha_ref, q_ref, k_ref, v_ref, wout_ref, bout_ref, out_ref):
    h = pl.program_id(1)
    scale = DIM_HEAD ** -0.5
    q = q_ref[0] * scale
    d = jax.lax.dot_general(
        q, k_ref[0], (((1,), (1,)), ((), ())),
        preferred_element_type=jnp.float32)

    ninf = jnp.float32(-jnp.inf)
    g1 = d[:, :_LANE]
    g2 = jnp.full_like(g1, ninf)
    g3 = g2
    for c in range(1, _NCOL):
        s = d[:, c * _LANE:(c + 1) * _LANE]
        t1 = jnp.minimum(g1, s)
        g1 = jnp.maximum(g1, s)
        t2 = jnp.minimum(g2, t1)
        g2 = jnp.maximum(g2, t1)
        g3 = jnp.maximum(g3, t2)
    m = jnp.max(g1, axis=-1, keepdims=True)
    head, nx1, nx2 = g1, g2, g3
    cur = m
    for _ in range(TOPK - 1):
        adv = head == cur
        head = jnp.where(adv, nx1, head)
        nx1 = jnp.where(adv, nx2, nx1)
        nx2 = jnp.where(adv, ninf, nx2)
        cur = jnp.max(head, axis=-1, keepdims=True)
    cnt = jnp.sum(jnp.where(d >= cur, 1.0, 0.0), axis=-1, keepdims=True)

    def _cond(carry):
        _, c = carry
        return jnp.any(c > 16.5)

    def _body(carry):
        cur, c = carry
        nxt = jnp.min(jnp.where(d > cur, d, jnp.float32(jnp.inf)),
                      axis=-1, keepdims=True)
        c2 = jnp.sum(jnp.where(d >= nxt, 1.0, 0.0), axis=-1, keepdims=True)
        take = (c > 16.5) & (c2 > 15.5)
        cur = jnp.where(take, nxt, cur)
        c = jnp.where(take, c2, jnp.where(c > 16.5, 16.0, c))
        return cur, c

    cur, _ = jax.lax.while_loop(_cond, _body, (cur, cnt))

    e = jnp.exp(d - m)
    em = jnp.where(d >= cur, e, 0.0)
    vv = v_ref[0]
    ev = jax.lax.dot_general(
        e, vv, (((1,), (0,)), ((), ())),
        preferred_element_type=jnp.float32)
    emv = jax.lax.dot_general(
        em, vv, (((1,), (0,)), ((), ())),
        preferred_element_type=jnp.float32)
    a = jnp.clip(alpha_ref[0, 0], 0.0, 1.0)
    out_h = (ev[:, :DIM_HEAD] * (a / ev[:, DIM_HEAD:])
             + emv[:, :DIM_HEAD] * ((1.0 - a) / emv[:, DIM_HEAD:]))
    part = jax.lax.dot_general(
        out_h, wout_ref[0], (((1,), (0,)), ((), ())),
        preferred_element_type=jnp.float32)

    @pl.when(h == 0)
    def _():
        out_ref[:] = part + bout_ref[:]

    @pl.when(h != 0)
    def _():
        out_ref[:] = out_ref[:] + part


@jax.jit
def _run(x, gamma, beta, W_qkv, W_out, b_out, alpha):
    x2 = x.reshape(SEQ, DIM)
    wout3 = W_out.reshape(HEADS, DIM_HEAD, DIM)

    q3, k3, v3 = pl.pallas_call(
        _ln_qkv_kernel,
        grid=(SEQ // BQ,),
        in_specs=[
            pl.BlockSpec((BQ, DIM), lambda i: (i, 0)),
            pl.BlockSpec((1, DIM), lambda i: (0, 0)),
            pl.BlockSpec((1, DIM), lambda i: (0, 0)),
            pl.BlockSpec((DIM, 3 * INNER), lambda i: (0, 0)),
        ],
        out_specs=[
            pl.BlockSpec((HEADS, BQ, DIM_HEAD), lambda i: (0, i, 0)),
            pl.BlockSpec((HEADS, BQ, DIM_HEAD), lambda i: (0, i, 0)),
            pl.BlockSpec((HEADS, BQ, 2 * DIM_HEAD), lambda i: (0, i, 0)),
        ],
        out_shape=[
            jax.ShapeDtypeStruct((HEADS, SEQ, DIM_HEAD), jnp.float32),
            jax.ShapeDtypeStruct((HEADS, SEQ, DIM_HEAD), jnp.float32),
            jax.ShapeDtypeStruct((HEADS, SEQ, 2 * DIM_HEAD), jnp.float32),
        ],
    )(x2, gamma.reshape(1, DIM), beta.reshape(1, DIM), W_qkv)

    out = pl.pallas_call(
        _attn_kernel,
        grid=(SEQ // BQ, HEADS),
        in_specs=[
            pl.BlockSpec((1, 1), lambda i, h: (0, 0)),
            pl.BlockSpec((1, BQ, DIM_HEAD), lambda i, h: (h, i, 0)),
            pl.BlockSpec((1, SEQ, DIM_HEAD), lambda i, h: (h, 0, 0)),
            pl.BlockSpec((1, SEQ, 2 * DIM_HEAD), lambda i, h: (h, 0, 0)),
            pl.BlockSpec((1, DIM_HEAD, DIM), lambda i, h: (h, 0, 0)),
            pl.BlockSpec((1, DIM), lambda i, h: (0, 0)),
        ],
        out_specs=pl.BlockSpec((BQ, DIM), lambda i, h: (i, 0)),
        out_shape=jax.ShapeDtypeStruct((SEQ, DIM), jnp.float32),
        compiler_params=pltpu.CompilerParams(
            dimension_semantics=("parallel", "arbitrary")),
    )(alpha.reshape(1, 1), q3, k3, v3, wout3, b_out.reshape(1, DIM))
    return out.reshape(1, SEQ, DIM)


def kernel(x, gamma, beta, W_qkv, W_out, b_out, alpha):
    return _run(x, gamma, beta, W_qkv, W_out, b_out, alpha)


# speedup vs baseline: 1.0204x; 1.0204x over previous
import jax
import jax.numpy as jnp
from jax.experimental import pallas as pl
from jax.experimental.pallas import tpu as pltpu

DIM = 768
HEADS = 12
DIM_HEAD = 64
TOPK = 16
INNER = HEADS * DIM_HEAD
SEQ = 2048
BQ = 1024
_LANE = 128
_NCOL = SEQ // _LANE


def _ln_qkv_kernel(x_ref, g_ref, b_ref, w_ref, q_ref, k_ref, v_ref):
    xb = x_ref[:]
    mu = jnp.mean(xb, axis=-1, keepdims=True)
    var = jnp.mean((xb - mu) ** 2, axis=-1, keepdims=True)
    xn = (xb - mu) * jax.lax.rsqrt(var + 1e-5)
    xn = xn * g_ref[:] + b_ref[:]
    qkv = jax.lax.dot_general(
        xn, w_ref[:], (((1,), (0,)), ((), ())),
        preferred_element_type=jnp.float32)
    ones = jnp.ones((BQ, DIM_HEAD), jnp.float32)
    for h in range(HEADS):
        q_ref[h] = qkv[:, h * DIM_HEAD:(h + 1) * DIM_HEAD]
        k_ref[h] = qkv[:, INNER + h * DIM_HEAD:INNER + (h + 1) * DIM_HEAD]
        vh = qkv[:, 2 * INNER + h * DIM_HEAD:2 * INNER + (h + 1) * DIM_HEAD]
        v_ref[h] = jnp.concatenate([vh, ones], axis=-1)


def _attn_kernel(alpha_ref, q_ref, k_ref, v_ref, wout_ref, bout_ref, out_ref):
    h = pl.program_id(1)
    scale = DIM_HEAD ** -0.5
    q = q_ref[0] * scale
    d = jax.lax.dot_general(
        q, k_ref[0], (((1,), (1,)), ((), ())),
        preferred_element_type=jnp.float32)

    ninf = jnp.float32(-jnp.inf)
    g1 = d[:, :_LANE]
    g2 = jnp.full_like(g1, ninf)
    g3 = g2
    for c in range(1, _NCOL):
        s = d[:, c * _LANE:(c + 1) * _LANE]
        t1 = jnp.minimum(g1, s)
        g1 = jnp.maximum(g1, s)
        t2 = jnp.minimum(g2, t1)
        g2 = jnp.maximum(g2, t1)
        g3 = jnp.maximum(g3, t2)
    m = jnp.max(g1, axis=-1, keepdims=True)
    cand = jnp.concatenate([g1, g2, g3], axis=-1)
    cur = m
    for _ in range(TOPK - 1):
        cur = jnp.max(jnp.where(cand < cur, cand, ninf),
                      axis=-1, keepdims=True)
    cnt = jnp.sum(jnp.where(d >= cur, 1.0, 0.0), axis=-1, keepdims=True)

    def _cond(carry):
        _, c = carry
        return jnp.any(c > 16.5)

    def _body(carry):
        cur, c = carry
        nxt = jnp.min(jnp.where(d > cur, d, jnp.float32(jnp.inf)),
                      axis=-1, keepdims=True)
        c2 = jnp.sum(jnp.where(d >= nxt, 1.0, 0.0), axis=-1, keepdims=True)
        take = (c > 16.5) & (c2 > 15.5)
        cur = jnp.where(take, nxt, cur)
        c = jnp.where(take, c2, jnp.where(c > 16.5, 16.0, c))
        return cur, c

    cur, _ = jax.lax.while_loop(_cond, _body, (cur, cnt))

    e = jnp.exp(d - m)
    em = jnp.where(d >= cur, e, 0.0)
    vv = v_ref[0]
    ev = jax.lax.dot_general(
        e, vv, (((1,), (0,)), ((), ())),
        preferred_element_type=jnp.float32)
    emv = jax.lax.dot_general(
        em, vv, (((1,), (0,)), ((), ())),
        preferred_element_type=jnp.float32)
    a = jnp.clip(alpha_ref[0, 0], 0.0, 1.0)
    out_h = (ev[:, :DIM_HEAD] * (a / ev[:, DIM_HEAD:])
             + emv[:, :DIM_HEAD] * ((1.0 - a) / emv[:, DIM_HEAD:]))
    part = jax.lax.dot_general(
        out_h, wout_ref[0], (((1,), (0,)), ((), ())),
        preferred_element_type=jnp.float32)

    @pl.when(h == 0)
    def _():
        out_ref[:] = part + bout_ref[:]

    @pl.when(h != 0)
    def _():
        out_ref[:] = out_ref[:] + part


@jax.jit
def _run(x, gamma, beta, W_qkv, W_out, b_out, alpha):
    x2 = x.reshape(SEQ, DIM)
    wout3 = W_out.reshape(HEADS, DIM_HEAD, DIM)

    q3, k3, v3 = pl.pallas_call(
        _ln_qkv_kernel,
        grid=(SEQ // BQ,),
        in_specs=[
            pl.BlockSpec((BQ, DIM), lambda i: (i, 0)),
            pl.BlockSpec((1, DIM), lambda i: (0, 0)),
            pl.BlockSpec((1, DIM), lambda i: (0, 0)),
            pl.BlockSpec((DIM, 3 * INNER), lambda i: (0, 0)),
        ],
        out_specs=[
            pl.BlockSpec((HEADS, BQ, DIM_HEAD), lambda i: (0, i, 0)),
            pl.BlockSpec((HEADS, BQ, DIM_HEAD), lambda i: (0, i, 0)),
            pl.BlockSpec((HEADS, BQ, 2 * DIM_HEAD), lambda i: (0, i, 0)),
        ],
        out_shape=[
            jax.ShapeDtypeStruct((HEADS, SEQ, DIM_HEAD), jnp.float32),
            jax.ShapeDtypeStruct((HEADS, SEQ, DIM_HEAD), jnp.float32),
            jax.ShapeDtypeStruct((HEADS, SEQ, 2 * DIM_HEAD), jnp.float32),
        ],
    )(x2, gamma.reshape(1, DIM), beta.reshape(1, DIM), W_qkv)

    out = pl.pallas_call(
        _attn_kernel,
        grid=(SEQ // BQ, HEADS),
        in_specs=[
            pl.BlockSpec((1, 1), lambda i, h: (0, 0)),
            pl.BlockSpec((1, BQ, DIM_HEAD), lambda i, h: (h, i, 0)),
            pl.BlockSpec((1, SEQ, DIM_HEAD), lambda i, h: (h, 0, 0)),
            pl.BlockSpec((1, SEQ, 2 * DIM_HEAD), lambda i, h: (h, 0, 0)),
            pl.BlockSpec((1, DIM_HEAD, DIM), lambda i, h: (h, 0, 0)),
            pl.BlockSpec((1, DIM), lambda i, h: (0, 0)),
        ],
        out_specs=pl.BlockSpec((BQ, DIM), lambda i, h: (i, 0)),
        out_shape=jax.ShapeDtypeStruct((SEQ, DIM), jnp.float32),
        compiler_params=pltpu.CompilerParams(
            dimension_semantics=("parallel", "arbitrary")),
    )(alpha.reshape(1, 1), q3, k3, v3, wout3, b_out.reshape(1, DIM))
    return out.reshape(1, SEQ, DIM)


def kernel(x, gamma, beta, W_qkv, W_out, b_out, alpha):
    return _run(x, gamma, beta, W_qkv, W_out, b_out, alpha)
